# Initial kernel scaffold; baseline (speedup 1.0000x reference)
import jax
import jax.numpy as jnp
from jax.experimental import pallas as pl


def kernel(feat0, feat1, feat2, feat3, feat4, rois):
    raise NotImplementedError("write your pallas kernel here")


# speedup vs baseline: 2.7507x; 2.7507x over previous
import functools

import jax
import jax.numpy as jnp
import numpy as np
from jax import lax
from jax.experimental import pallas as pl
from jax.experimental.pallas import tpu as pltpu
from jax.experimental.pallas import tpu_sc as plsc

IMG = 512.0
SIZES = [(128, 128), (64, 64), (32, 32), (16, 16), (8, 8)]
SCALES = [0.25, 0.125, 0.0625, 0.03125, 0.015625]
OUT = 7
SR = 2
B = 2
C = 256
N = 1000
R = B * N
G = OUT * SR
T = int(np.sum([h * w for (h, w) in SIZES]))
BT = B * T

NC, NS = 2, 16
NW = NC * NS
ROWS = R * OUT * OUT
CH = 8
PER_W = 3072
ROWS_PAD = NW * PER_W
NCH = PER_W // CH
IPW = NCH
NBUF = 3


def _axis_interp(coord, size_f, size_i):
    valid = (coord >= -1.0) & (coord <= size_f)
    c = jnp.maximum(coord, 0.0)
    low = c.astype(jnp.int32)
    cond = low >= (size_i - 1)
    high = jnp.where(cond, size_i - 1, low + 1)
    low = jnp.where(cond, size_i - 1, low)
    c = jnp.where(cond, low.astype(coord.dtype), c)
    l = c - low.astype(coord.dtype)
    return valid, low, high, l, 1.0 - l


def _tap_lists(rois):
    areas = [h * w for (h, w) in SIZES]
    offs = np.concatenate([[0], np.cumsum(areas)[:-1]]).astype(np.int32)
    boxes = rois.reshape(R, 4)
    bidx = jnp.repeat(jnp.arange(B, dtype=jnp.int32), N)
    bw = boxes[:, 2] - boxes[:, 0]
    bh = boxes[:, 3] - boxes[:, 1]
    s = jnp.sqrt(jnp.maximum(bw * bh, 1e-12))
    lvl = jnp.floor(4.0 + jnp.log2(s / 224.0) + 1e-6)
    lvl = jnp.clip(lvl, 2.0, 6.0).astype(jnp.int32) - 2
    sc = jnp.array(SCALES, dtype=jnp.float32)[lvl]
    Hf = jnp.array([h for (h, w) in SIZES], dtype=jnp.float32)[lvl]
    Wf = jnp.array([w for (h, w) in SIZES], dtype=jnp.float32)[lvl]
    Hi = jnp.array([h for (h, w) in SIZES], dtype=jnp.int32)[lvl]
    Wi = jnp.array([w for (h, w) in SIZES], dtype=jnp.int32)[lvl]
    off = jnp.array(offs, dtype=jnp.int32)[lvl]
    rs_w = boxes[:, 0] * sc
    rs_h = boxes[:, 1] * sc
    roi_w = jnp.maximum(boxes[:, 2] * sc - rs_w, 1.0)
    roi_h = jnp.maximum(boxes[:, 3] * sc - rs_h, 1.0)
    bin_w = roi_w / OUT
    bin_h = roi_h / OUT
    g = np.array([p + (i + 0.5) / SR for p in range(OUT) for i in range(SR)],
                 dtype=np.float32)
    ys = rs_h[:, None] + g[None, :] * bin_h[:, None]
    xs = rs_w[:, None] + g[None, :] * bin_w[:, None]
    vy, yl, yh, ly, hy = _axis_interp(ys, Hf[:, None], Hi[:, None])
    vx, xl, xh, lx, hx = _axis_interp(xs, Wf[:, None], Wi[:, None])
    valid = (vy[:, :, None] & vx[:, None, :]).astype(jnp.float32) * 0.25
    w1 = hy[:, :, None] * hx[:, None, :] * valid
    w2 = hy[:, :, None] * lx[:, None, :] * valid
    w3 = ly[:, :, None] * hx[:, None, :] * valid
    w4 = ly[:, :, None] * lx[:, None, :] * valid
    base = (bidx * T + off)[:, None, None]
    Wst = Wi[:, None, None]
    i1 = base + yl[:, :, None] * Wst + xl[:, None, :]
    i2 = base + yl[:, :, None] * Wst + xh[:, None, :]
    i3 = base + yh[:, :, None] * Wst + xl[:, None, :]
    i4 = base + yh[:, :, None] * Wst + xh[:, None, :]
    idx = jnp.stack([i1, i2, i3, i4], axis=-1)
    wts = jnp.stack([w1, w2, w3, w4], axis=-1)
    idx = idx.reshape(R, OUT, SR, OUT, SR, 4).transpose(0, 1, 3, 2, 4, 5)
    wts = wts.reshape(R, OUT, SR, OUT, SR, 4).transpose(0, 1, 3, 2, 4, 5)
    idx = idx.reshape(R * OUT * OUT, 16)
    wts = wts.reshape(R * OUT * OUT, 16)
    pad = ROWS_PAD - ROWS
    idx = jnp.pad(idx, ((0, pad), (0, 0)))
    wts = jnp.pad(wts, ((0, pad), (0, 0)))
    return idx.reshape(-1, 128), wts.reshape(-1, 128)


def _sc_body(table, idx_hbm, wts_hbm, out_hbm, *scratch):
    rows_v = scratch[0:NBUF]
    out_v = scratch[NBUF:2 * NBUF]
    idx_v = scratch[2 * NBUF:3 * NBUF]
    wts_v = scratch[3 * NBUF:4 * NBUF]
    sem_g = scratch[4 * NBUF:5 * NBUF]
    sem_o = scratch[5 * NBUF:6 * NBUF]
    sem_i = scratch[6 * NBUF:7 * NBUF]
    sem_w = scratch[7 * NBUF:8 * NBUF]
    wid = lax.axis_index("s") * NC + lax.axis_index("c")
    irow0 = wid * IPW
    orow0 = wid * PER_W

    def load_idx(c, b):
        pltpu.async_copy(idx_hbm.at[irow0 + c], idx_v[b], sem_i[b])
        pltpu.async_copy(wts_hbm.at[irow0 + c], wts_v[b], sem_w[b])

    def wait_idx(c, b):
        pltpu.make_async_copy(idx_hbm.at[irow0 + c], idx_v[b], sem_i[b]).wait()
        pltpu.make_async_copy(wts_hbm.at[irow0 + c], wts_v[b], sem_w[b]).wait()

    def start_gather(c, b):
        pltpu.async_copy(table.at[idx_v[b]], rows_v[b], sem_g[b])

    def wait_gather(c, b):
        pltpu.make_async_copy(table.at[idx_v[b]], rows_v[b], sem_g[b]).wait()

    def start_store(c, b):
        pltpu.async_copy(out_v[b], out_hbm.at[pl.ds(orow0 + c * CH, CH)],
                         sem_o[b])

    def wait_store(c, b):
        pltpu.make_async_copy(out_v[b], out_hbm.at[pl.ds(orow0 + c * CH, CH)],
                              sem_o[b]).wait()

    def compute(b):
        def row_body(o, carry):
            wv = wts_v[b][pl.ds(o * 16, 16)]
            for j in range(16):
                acc = jnp.zeros((16,), jnp.float32)
                for t in range(16):
                    acc = acc + wv[t] * rows_v[b][o * 16 + t, pl.ds(j * 16, 16)]
                out_v[b][o, pl.ds(j * 16, 16)] = acc
            return carry
        lax.fori_loop(0, CH, row_body, 0)

    for b in range(NBUF):
        load_idx(b, b)
    for b in range(2):
        wait_idx(b, b)
        start_gather(b, b)

    def group(gg, carry):
        g0 = gg * NBUF
        for b in range(NBUF):
            c = g0 + b
            bp2 = (b + 2) % NBUF

            @pl.when(c + 2 < NCH)
            def _():
                wait_idx(c + 2, bp2)
                start_gather(c + 2, bp2)

            wait_gather(c, b)

            @pl.when(c >= NBUF)
            def _():
                wait_store(c - NBUF, b)

            compute(b)
            start_store(c, b)

            @pl.when(c + NBUF < NCH)
            def _():
                load_idx(c + NBUF, b)
        return carry

    lax.fori_loop(0, NCH // NBUF, group, 0)

    for b in range(NBUF):
        wait_store(NCH - NBUF + b, (NCH - NBUF + b) % NBUF)


@functools.cache
def _sc_gather():
    return pl.kernel(
        _sc_body,
        out_type=jax.ShapeDtypeStruct((ROWS_PAD, C), jnp.float32),
        mesh=plsc.VectorSubcoreMesh(core_axis_name="c", subcore_axis_name="s",
                                    num_cores=NC, num_subcores=NS),
        scratch_types=(
            [pltpu.VMEM((CH * 16, C), jnp.float32)] * NBUF
            + [pltpu.VMEM((CH, C), jnp.float32)] * NBUF
            + [pltpu.VMEM((128,), jnp.int32)] * NBUF
            + [pltpu.VMEM((128,), jnp.float32)] * NBUF
            + [pltpu.SemaphoreType.DMA] * (4 * NBUF)
        ),
    )


def kernel(feat0, feat1, feat2, feat3, feat4, rois):
    feats = [feat0, feat1, feat2, feat3, feat4]
    table = jnp.concatenate([f.reshape(B, C, -1) for f in feats], axis=2)
    table = table.transpose(0, 2, 1).reshape(BT, C)
    idx, wts = _tap_lists(rois)
    out = _sc_gather()(table, idx, wts)
    out = out[:ROWS].reshape(R, OUT, OUT, C).transpose(0, 3, 1, 2)
    return out


# speedup vs baseline: 3.1019x; 1.1277x over previous
import functools

import jax
import jax.numpy as jnp
import numpy as np
from jax import lax
from jax.experimental import pallas as pl
from jax.experimental.pallas import tpu as pltpu
from jax.experimental.pallas import tpu_sc as plsc

IMG = 512.0
SIZES = [(128, 128), (64, 64), (32, 32), (16, 16), (8, 8)]
SCALES = [0.25, 0.125, 0.0625, 0.03125, 0.015625]
OUT = 7
SR = 2
B = 2
C = 256
N = 1000
R = B * N
G = OUT * SR
T = int(np.sum([h * w for (h, w) in SIZES]))
BT = B * T

NC, NS = 2, 16
NW = NC * NS
ROWS = R * OUT * OUT
CH = 8
PER_W = 3072
ROWS_PAD = NW * PER_W
NCH = PER_W // CH
IPW = NCH
NBUF = 3


def _axis_interp(coord, size_f, size_i):
    valid = (coord >= -1.0) & (coord <= size_f)
    c = jnp.maximum(coord, 0.0)
    low = c.astype(jnp.int32)
    cond = low >= (size_i - 1)
    high = jnp.where(cond, size_i - 1, low + 1)
    low = jnp.where(cond, size_i - 1, low)
    c = jnp.where(cond, low.astype(coord.dtype), c)
    l = c - low.astype(coord.dtype)
    return valid, low, high, l, 1.0 - l


def _tap_lists(rois):
    areas = [h * w for (h, w) in SIZES]
    offs = np.concatenate([[0], np.cumsum(areas)[:-1]]).astype(np.int32)
    boxes = rois.reshape(R, 4)
    bidx = jnp.repeat(jnp.arange(B, dtype=jnp.int32), N)
    bw = boxes[:, 2] - boxes[:, 0]
    bh = boxes[:, 3] - boxes[:, 1]
    s = jnp.sqrt(jnp.maximum(bw * bh, 1e-12))
    lvl = jnp.floor(4.0 + jnp.log2(s / 224.0) + 1e-6)
    lvl = jnp.clip(lvl, 2.0, 6.0).astype(jnp.int32) - 2
    sc = jnp.array(SCALES, dtype=jnp.float32)[lvl]
    Hf = jnp.array([h for (h, w) in SIZES], dtype=jnp.float32)[lvl]
    Wf = jnp.array([w for (h, w) in SIZES], dtype=jnp.float32)[lvl]
    Hi = jnp.array([h for (h, w) in SIZES], dtype=jnp.int32)[lvl]
    Wi = jnp.array([w for (h, w) in SIZES], dtype=jnp.int32)[lvl]
    off = jnp.array(offs, dtype=jnp.int32)[lvl]
    rs_w = boxes[:, 0] * sc
    rs_h = boxes[:, 1] * sc
    roi_w = jnp.maximum(boxes[:, 2] * sc - rs_w, 1.0)
    roi_h = jnp.maximum(boxes[:, 3] * sc - rs_h, 1.0)
    bin_w = roi_w / OUT
    bin_h = roi_h / OUT
    g = np.array([p + (i + 0.5) / SR for p in range(OUT) for i in range(SR)],
                 dtype=np.float32)
    ys = rs_h[:, None] + g[None, :] * bin_h[:, None]
    xs = rs_w[:, None] + g[None, :] * bin_w[:, None]
    vy, yl, yh, ly, hy = _axis_interp(ys, Hf[:, None], Hi[:, None])
    vx, xl, xh, lx, hx = _axis_interp(xs, Wf[:, None], Wi[:, None])
    valid = (vy[:, :, None] & vx[:, None, :]).astype(jnp.float32) * 0.25
    w1 = hy[:, :, None] * hx[:, None, :] * valid
    w2 = hy[:, :, None] * lx[:, None, :] * valid
    w3 = ly[:, :, None] * hx[:, None, :] * valid
    w4 = ly[:, :, None] * lx[:, None, :] * valid
    base = (bidx * T + off)[:, None, None]
    Wst = Wi[:, None, None]
    i1 = base + yl[:, :, None] * Wst + xl[:, None, :]
    i2 = base + yl[:, :, None] * Wst + xh[:, None, :]
    i3 = base + yh[:, :, None] * Wst + xl[:, None, :]
    i4 = base + yh[:, :, None] * Wst + xh[:, None, :]
    idx = jnp.stack([i1, i2, i3, i4], axis=-1)
    wts = jnp.stack([w1, w2, w3, w4], axis=-1)
    idx = idx.reshape(R, OUT, SR, OUT, SR, 4).transpose(0, 1, 3, 2, 4, 5)
    wts = wts.reshape(R, OUT, SR, OUT, SR, 4).transpose(0, 1, 3, 2, 4, 5)
    idx = idx.reshape(R * OUT * OUT, 16)
    wts = wts.reshape(R * OUT * OUT, 16)
    pad = ROWS_PAD - ROWS
    idx = jnp.pad(idx, ((0, pad), (0, 0)))
    wts = jnp.pad(wts, ((0, pad), (0, 0)))
    return idx.reshape(-1, 128), wts.reshape(-1, 128)


def _sc_body(table, idx_hbm, wts_hbm, out_hbm, *scratch):
    rows_v = scratch[0:NBUF]
    out_v = scratch[NBUF:2 * NBUF]
    idx_v = scratch[2 * NBUF:3 * NBUF]
    wts_v = scratch[3 * NBUF:4 * NBUF]
    sem_g = scratch[4 * NBUF:5 * NBUF]
    sem_o = scratch[5 * NBUF:6 * NBUF]
    sem_i = scratch[6 * NBUF:7 * NBUF]
    sem_w = scratch[7 * NBUF:8 * NBUF]
    wid = lax.axis_index("s") * NC + lax.axis_index("c")
    irow0 = wid * IPW
    orow0 = wid * PER_W

    def load_idx(c, b):
        pltpu.async_copy(idx_hbm.at[irow0 + c], idx_v[b], sem_i[b])
        pltpu.async_copy(wts_hbm.at[irow0 + c], wts_v[b], sem_w[b])

    def wait_idx(c, b):
        pltpu.make_async_copy(idx_hbm.at[irow0 + c], idx_v[b], sem_i[b]).wait()
        pltpu.make_async_copy(wts_hbm.at[irow0 + c], wts_v[b], sem_w[b]).wait()

    def start_gather(c, b):
        pltpu.async_copy(table.at[idx_v[b]], rows_v[b], sem_g[b])

    def wait_gather(c, b):
        pltpu.make_async_copy(table.at[idx_v[b]], rows_v[b], sem_g[b]).wait()

    def start_store(c, b):
        pltpu.async_copy(out_v[b], out_hbm.at[pl.ds(orow0 + c * CH, CH)],
                         sem_o[b])

    def wait_store(c, b):
        pltpu.make_async_copy(out_v[b], out_hbm.at[pl.ds(orow0 + c * CH, CH)],
                              sem_o[b]).wait()

    def compute(b):
        mask_hi = jnp.int32(-65536)
        def row_body(o, carry):
            wv = wts_v[b][pl.ds(o * 16, 16)]
            for j in range(8):
                acc_e = jnp.zeros((16,), jnp.float32)
                acc_o = jnp.zeros((16,), jnp.float32)
                for t in range(16):
                    u = rows_v[b][o * 16 + t, pl.ds(j * 16, 16)]
                    xe = lax.bitcast_convert_type(u << 16, jnp.float32)
                    xo = lax.bitcast_convert_type(u & mask_hi, jnp.float32)
                    acc_e = acc_e + wv[t] * xe
                    acc_o = acc_o + wv[t] * xo
                out_v[b][o, pl.ds(j * 16, 16)] = acc_e
                out_v[b][o, pl.ds(128 + j * 16, 16)] = acc_o
            return carry
        lax.fori_loop(0, CH, row_body, 0)

    for b in range(NBUF):
        load_idx(b, b)
    for b in range(2):
        wait_idx(b, b)
        start_gather(b, b)

    def group(gg, carry):
        g0 = gg * NBUF
        for b in range(NBUF):
            c = g0 + b
            bp2 = (b + 2) % NBUF

            @pl.when(c + 2 < NCH)
            def _():
                wait_idx(c + 2, bp2)
                start_gather(c + 2, bp2)

            wait_gather(c, b)

            @pl.when(c >= NBUF)
            def _():
                wait_store(c - NBUF, b)

            compute(b)
            start_store(c, b)

            @pl.when(c + NBUF < NCH)
            def _():
                load_idx(c + NBUF, b)
        return carry

    lax.fori_loop(0, NCH // NBUF, group, 0)

    for b in range(NBUF):
        wait_store(NCH - NBUF + b, (NCH - NBUF + b) % NBUF)


@functools.cache
def _sc_gather():
    return pl.kernel(
        _sc_body,
        out_type=jax.ShapeDtypeStruct((ROWS_PAD, C), jnp.float32),
        mesh=plsc.VectorSubcoreMesh(core_axis_name="c", subcore_axis_name="s",
                                    num_cores=NC, num_subcores=NS),
        scratch_types=(
            [pltpu.VMEM((CH * 16, C // 2), jnp.int32)] * NBUF
            + [pltpu.VMEM((CH, C), jnp.float32)] * NBUF
            + [pltpu.VMEM((128,), jnp.int32)] * NBUF
            + [pltpu.VMEM((128,), jnp.float32)] * NBUF
            + [pltpu.SemaphoreType.DMA] * (4 * NBUF)
        ),
    )


def kernel(feat0, feat1, feat2, feat3, feat4, rois):
    feats = [feat0, feat1, feat2, feat3, feat4]
    table = jnp.concatenate([f.reshape(B, C, -1) for f in feats], axis=2)
    table = table.transpose(0, 2, 1).reshape(BT, C)
    table = table.reshape(BT, 2, C // 2).transpose(0, 2, 1)
    table = lax.bitcast_convert_type(table.astype(jnp.bfloat16), jnp.int32)
    idx, wts = _tap_lists(rois)
    out = _sc_gather()(table, idx, wts)
    out = out[:ROWS].reshape(R, OUT, OUT, C).transpose(0, 3, 1, 2)
    return out


# speedup vs baseline: 3.1066x; 1.0015x over previous
import functools

import jax
import jax.numpy as jnp
import numpy as np
from jax import lax
from jax.experimental import pallas as pl
from jax.experimental.pallas import tpu as pltpu
from jax.experimental.pallas import tpu_sc as plsc

IMG = 512.0
SIZES = [(128, 128), (64, 64), (32, 32), (16, 16), (8, 8)]
SCALES = [0.25, 0.125, 0.0625, 0.03125, 0.015625]
OUT = 7
SR = 2
B = 2
C = 256
N = 1000
R = B * N
G = OUT * SR
T = int(np.sum([h * w for (h, w) in SIZES]))
BT = B * T

NC, NS = 2, 16
NW = NC * NS
ROWS = R * OUT * OUT
CH = 8
PER_W = 3072
ROWS_PAD = NW * PER_W
NCH = PER_W // CH
IPW = NCH
NBUF = 3


def _axis_interp(coord, size_f, size_i):
    valid = (coord >= -1.0) & (coord <= size_f)
    c = jnp.maximum(coord, 0.0)
    low = c.astype(jnp.int32)
    cond = low >= (size_i - 1)
    high = jnp.where(cond, size_i - 1, low + 1)
    low = jnp.where(cond, size_i - 1, low)
    c = jnp.where(cond, low.astype(coord.dtype), c)
    l = c - low.astype(coord.dtype)
    return valid, low, high, l, 1.0 - l


def _tap_lists(rois):
    areas = [h * w for (h, w) in SIZES]
    offs = np.concatenate([[0], np.cumsum(areas)[:-1]]).astype(np.int32)
    boxes = rois.reshape(R, 4)
    bidx = jnp.repeat(jnp.arange(B, dtype=jnp.int32), N)
    bw = boxes[:, 2] - boxes[:, 0]
    bh = boxes[:, 3] - boxes[:, 1]
    s = jnp.sqrt(jnp.maximum(bw * bh, 1e-12))
    lvl = jnp.floor(4.0 + jnp.log2(s / 224.0) + 1e-6)
    lvl = jnp.clip(lvl, 2.0, 6.0).astype(jnp.int32) - 2
    sc = jnp.array(SCALES, dtype=jnp.float32)[lvl]
    Hf = jnp.array([h for (h, w) in SIZES], dtype=jnp.float32)[lvl]
    Wf = jnp.array([w for (h, w) in SIZES], dtype=jnp.float32)[lvl]
    Hi = jnp.array([h for (h, w) in SIZES], dtype=jnp.int32)[lvl]
    Wi = jnp.array([w for (h, w) in SIZES], dtype=jnp.int32)[lvl]
    off = jnp.array(offs, dtype=jnp.int32)[lvl]
    rs_w = boxes[:, 0] * sc
    rs_h = boxes[:, 1] * sc
    roi_w = jnp.maximum(boxes[:, 2] * sc - rs_w, 1.0)
    roi_h = jnp.maximum(boxes[:, 3] * sc - rs_h, 1.0)
    bin_w = roi_w / OUT
    bin_h = roi_h / OUT
    g = np.array([p + (i + 0.5) / SR for p in range(OUT) for i in range(SR)],
                 dtype=np.float32)
    ys = rs_h[:, None] + g[None, :] * bin_h[:, None]
    xs = rs_w[:, None] + g[None, :] * bin_w[:, None]
    vy, yl, yh, ly, hy = _axis_interp(ys, Hf[:, None], Hi[:, None])
    vx, xl, xh, lx, hx = _axis_interp(xs, Wf[:, None], Wi[:, None])
    def _yterm(a):
        return a.reshape(R, OUT, 1, SR, 1)
    def _xterm(a):
        return a.reshape(R, 1, OUT, 1, SR)
    vy, yl, yh, ly, hy = map(_yterm, (vy, yl, yh, ly, hy))
    vx, xl, xh, lx, hx = map(_xterm, (vx, xl, xh, lx, hx))
    valid = (vy & vx).astype(jnp.float32) * 0.25
    w1 = hy * hx * valid
    w2 = hy * lx * valid
    w3 = ly * hx * valid
    w4 = ly * lx * valid
    base = (bidx * T + off).reshape(R, 1, 1, 1, 1)
    Wst = Wi.reshape(R, 1, 1, 1, 1)
    i1 = base + yl * Wst + xl
    i2 = base + yl * Wst + xh
    i3 = base + yh * Wst + xl
    i4 = base + yh * Wst + xh
    idx = jnp.stack([i1, i2, i3, i4], axis=-1)
    wts = jnp.stack([w1, w2, w3, w4], axis=-1)
    idx = idx.reshape(R * OUT * OUT, 16)
    wts = wts.reshape(R * OUT * OUT, 16)
    pad = ROWS_PAD - ROWS
    idx = jnp.pad(idx, ((0, pad), (0, 0)))
    wts = jnp.pad(wts, ((0, pad), (0, 0)))
    return idx.reshape(-1, 128), wts.reshape(-1, 128)


def _sc_body(table, idx_hbm, wts_hbm, out_hbm, *scratch):
    rows_v = scratch[0:NBUF]
    out_v = scratch[NBUF:2 * NBUF]
    idx_v = scratch[2 * NBUF:3 * NBUF]
    wts_v = scratch[3 * NBUF:4 * NBUF]
    sem_g = scratch[4 * NBUF:5 * NBUF]
    sem_o = scratch[5 * NBUF:6 * NBUF]
    sem_i = scratch[6 * NBUF:7 * NBUF]
    sem_w = scratch[7 * NBUF:8 * NBUF]
    wid = lax.axis_index("s") * NC + lax.axis_index("c")
    irow0 = wid * IPW
    orow0 = wid * PER_W

    def load_idx(c, b):
        pltpu.async_copy(idx_hbm.at[irow0 + c], idx_v[b], sem_i[b])
        pltpu.async_copy(wts_hbm.at[irow0 + c], wts_v[b], sem_w[b])

    def wait_idx(c, b):
        pltpu.make_async_copy(idx_hbm.at[irow0 + c], idx_v[b], sem_i[b]).wait()
        pltpu.make_async_copy(wts_hbm.at[irow0 + c], wts_v[b], sem_w[b]).wait()

    def start_gather(c, b):
        pltpu.async_copy(table.at[idx_v[b]], rows_v[b], sem_g[b])

    def wait_gather(c, b):
        pltpu.make_async_copy(table.at[idx_v[b]], rows_v[b], sem_g[b]).wait()

    def start_store(c, b):
        pltpu.async_copy(out_v[b], out_hbm.at[pl.ds(orow0 + c * CH, CH)],
                         sem_o[b])

    def wait_store(c, b):
        pltpu.make_async_copy(out_v[b], out_hbm.at[pl.ds(orow0 + c * CH, CH)],
                              sem_o[b]).wait()

    def compute(b):
        mask_hi = jnp.int32(-65536)
        def row_body(o, carry):
            wv = wts_v[b][pl.ds(o * 16, 16)]
            for j in range(8):
                acc_e = jnp.zeros((16,), jnp.float32)
                acc_o = jnp.zeros((16,), jnp.float32)
                for t in range(16):
                    u = rows_v[b][o * 16 + t, pl.ds(j * 16, 16)]
                    xe = lax.bitcast_convert_type(u << 16, jnp.float32)
                    xo = lax.bitcast_convert_type(u & mask_hi, jnp.float32)
                    acc_e = acc_e + wv[t] * xe
                    acc_o = acc_o + wv[t] * xo
                out_v[b][o, pl.ds(j * 16, 16)] = acc_e
                out_v[b][o, pl.ds(128 + j * 16, 16)] = acc_o
            return carry
        lax.fori_loop(0, CH, row_body, 0)

    for b in range(NBUF):
        load_idx(b, b)
    for b in range(2):
        wait_idx(b, b)
        start_gather(b, b)

    def group(gg, carry):
        g0 = gg * NBUF
        for b in range(NBUF):
            c = g0 + b
            bp2 = (b + 2) % NBUF

            @pl.when(c + 2 < NCH)
            def _():
                wait_idx(c + 2, bp2)
                start_gather(c + 2, bp2)

            wait_gather(c, b)

            @pl.when(c >= NBUF)
            def _():
                wait_store(c - NBUF, b)

            compute(b)
            start_store(c, b)

            @pl.when(c + NBUF < NCH)
            def _():
                load_idx(c + NBUF, b)
        return carry

    lax.fori_loop(0, NCH // NBUF, group, 0)

    for b in range(NBUF):
        wait_store(NCH - NBUF + b, (NCH - NBUF + b) % NBUF)


@functools.cache
def _sc_gather():
    return pl.kernel(
        _sc_body,
        out_type=jax.ShapeDtypeStruct((ROWS_PAD, C), jnp.float32),
        mesh=plsc.VectorSubcoreMesh(core_axis_name="c", subcore_axis_name="s",
                                    num_cores=NC, num_subcores=NS),
        scratch_types=(
            [pltpu.VMEM((CH * 16, C // 2), jnp.int32)] * NBUF
            + [pltpu.VMEM((CH, C), jnp.float32)] * NBUF
            + [pltpu.VMEM((128,), jnp.int32)] * NBUF
            + [pltpu.VMEM((128,), jnp.float32)] * NBUF
            + [pltpu.SemaphoreType.DMA] * (4 * NBUF)
        ),
    )


def kernel(feat0, feat1, feat2, feat3, feat4, rois):
    feats = [feat0, feat1, feat2, feat3, feat4]
    table = jnp.concatenate([f.reshape(B, C, -1) for f in feats], axis=2)
    table = table.transpose(0, 2, 1).reshape(BT, C)
    table = table.reshape(BT, 2, C // 2).transpose(0, 2, 1)
    table = lax.bitcast_convert_type(table.astype(jnp.bfloat16), jnp.int32)
    idx, wts = _tap_lists(rois)
    out = _sc_gather()(table, idx, wts)
    out = out[:ROWS].reshape(R, OUT, OUT, C).transpose(0, 3, 1, 2)
    return out
